# baseline (speedup 1.0000x reference)
import jax
import jax.numpy as jnp
from jax.experimental import pallas as pl
from jax.experimental.pallas import tpu as pltpu

_IN = 1024
_OUT = 32768
_K = 256
_B = 8
_BLOCK_N = 2048
_NBLK = _OUT // _BLOCK_N
_NBOUND = 32
_SPACING = 1024


def _float_key(t):
    i = jax.lax.bitcast_convert_type(t, jnp.int32)
    return jnp.where(i >= 0, i, i ^ jnp.int32(0x7FFFFFFF))


def _key_float(k):
    i = jnp.where(k >= 0, k, k ^ jnp.int32(0x7FFFFFFF))
    return jax.lax.bitcast_convert_type(i, jnp.float32)


def _count_ge(key, cand):
    return jnp.sum((key >= cand).astype(jnp.int32), axis=1, keepdims=True)


def _kth_largest_full(key, k):
    msb = jnp.int32(-2147483648)
    prefix_u = jnp.zeros((key.shape[0], 1), jnp.int32)
    for bit in range(31, -1, -1):
        bitval = (1 << bit) if bit < 31 else -(1 << 31)
        cand_u = prefix_u | jnp.int32(bitval)
        cnt = _count_ge(key, cand_u ^ msb)
        prefix_u = jnp.where(cnt >= k, cand_u, prefix_u)
    return prefix_u ^ msb


def _wta_kernel(x_ref, w_ref, b_ref, out_ref, key_ref, hist_ref, tau_ref,
                th_ref, mb_ref):
    step = pl.program_id(0)
    t_blk = jax.lax.dot_general(
        x_ref[...], w_ref[...],
        (((1,), (1,)), ((), ())),
        preferred_element_type=jnp.float32,
    ) + b_ref[...]
    key_blk = _float_key(t_blk)
    key_ref[:, pl.ds(step * _BLOCK_N, _BLOCK_N)] = key_blk

    @pl.when(step == 0)
    def _():
        est = _kth_largest_full(key_blk, _K // _NBLK)
        tau_ref[...] = jnp.broadcast_to(est & jnp.int32(-_SPACING),
                                        (_B, 128))

    tau = tau_ref[:, :1]
    lane = jax.lax.broadcasted_iota(jnp.int32, (_B, _NBOUND), 1)
    acc = jnp.zeros((_B, _NBOUND), jnp.int32)
    for j in range(_NBOUND):
        bj = tau + jnp.int32((_NBOUND // 2 - j) * _SPACING)
        acc = jnp.where(lane == j, _count_ge(key_blk, bj), acc)
    hist_ref[step] = acc

    @pl.when(step == _NBLK - 1)
    def _():
        key = key_ref[...]
        tau0 = tau_ref[:, :1]
        hist = jnp.sum(hist_ref[...], axis=0)
        ok = (hist >= _K).astype(jnp.int32)
        jstar = _NBOUND - jnp.sum(ok, axis=1, keepdims=True)
        miss = jnp.any((jstar < 1) | (jstar > _NBOUND - 1))
        m = tau0 + (_NBOUND // 2 - jstar) * jnp.int32(_SPACING)

        @pl.when(jnp.logical_not(miss))
        def _refine():
            prefix = m
            for bit in range(9, -1, -1):
                cand = prefix + jnp.int32(1 << bit)
                cnt = _count_ge(key, cand)
                prefix = jnp.where(cnt >= _K, cand, prefix)
            th_ref[...] = jnp.broadcast_to(prefix, (_B, 128))

        @pl.when(miss)
        def _full():
            th_ref[...] = jnp.broadcast_to(_kth_largest_full(key, _K),
                                           (_B, 128))

        thresh = th_ref[:, :1]
        gt = key > thresh
        eq = key == thresh
        n_ge = jnp.sum((gt | eq).astype(jnp.int32), axis=1, keepdims=True)
        col = jax.lax.broadcasted_iota(jnp.int32, (_B, _OUT), 1)
        mb_ref[...] = jnp.full((_B, 128), _OUT, jnp.int32)

        @pl.when(jnp.any(n_ge > _K))
        def _tie():
            r = _K - (n_ge - jnp.sum(eq.astype(jnp.int32), axis=1,
                                     keepdims=True))
            mpref = jnp.zeros((_B, 1), jnp.int32)
            for bit in range(15, -1, -1):
                cand = mpref | jnp.int32(1 << bit)
                cntc = jnp.sum((eq & (col < cand)).astype(jnp.int32),
                               axis=1, keepdims=True)
                mpref = jnp.where(cntc <= r, cand, mpref)
            mb_ref[...] = jnp.broadcast_to(mpref, (_B, 128))

        accept = gt | (eq & (col < mb_ref[:, :1]))
        kmin = jnp.int32(-2147483648)
        pooled_k = jnp.max(jnp.where(accept, key, kmin), axis=0,
                           keepdims=True)
        out_ref[...] = jnp.where(pooled_k == kmin, jnp.float32(0.0),
                                 _key_float(pooled_k))


def kernel(inputs, W, b):
    out = pl.pallas_call(
        _wta_kernel,
        grid=(_NBLK,),
        in_specs=[
            pl.BlockSpec((_B, _IN), lambda i: (0, 0)),
            pl.BlockSpec((_BLOCK_N, _IN), lambda i: (i, 0)),
            pl.BlockSpec((1, _BLOCK_N), lambda i: (0, i)),
        ],
        out_specs=pl.BlockSpec((1, _OUT), lambda i: (0, 0)),
        out_shape=jax.ShapeDtypeStruct((1, _OUT), jnp.float32),
        scratch_shapes=[
            pltpu.VMEM((_B, _OUT), jnp.int32),
            pltpu.VMEM((_NBLK, _B, _NBOUND), jnp.int32),
            pltpu.VMEM((_B, 128), jnp.int32),
            pltpu.VMEM((_B, 128), jnp.int32),
            pltpu.VMEM((_B, 128), jnp.int32),
        ],
    )(inputs, W, b.reshape(1, _OUT))
    return out.reshape(_OUT)


# speedup vs baseline: 1.2391x; 1.2391x over previous
import jax
import jax.numpy as jnp
from jax.experimental import pallas as pl
from jax.experimental.pallas import tpu as pltpu

_IN = 1024
_OUT = 32768
_K = 256
_B = 8
_BLOCK_N = 2048
_NBLK = _OUT // _BLOCK_N
_NBOUND = 32
_SPACING = 2048
_SUB = 512
_SUBRANK = 4


def _float_key(t):
    i = jax.lax.bitcast_convert_type(t, jnp.int32)
    return jnp.where(i >= 0, i, i ^ jnp.int32(0x7FFFFFFF))


def _key_float(k):
    i = jnp.where(k >= 0, k, k ^ jnp.int32(0x7FFFFFFF))
    return jax.lax.bitcast_convert_type(i, jnp.float32)


def _count_ge(key, cand):
    return jnp.sum((key >= cand).astype(jnp.int32), axis=1, keepdims=True)


def _kth_largest(key, k, lo_bit=0):
    msb = jnp.int32(-2147483648)
    prefix_u = jnp.zeros((key.shape[0], 1), jnp.int32)
    for bit in range(31, lo_bit - 1, -1):
        bitval = (1 << bit) if bit < 31 else -(1 << 31)
        cand_u = prefix_u | jnp.int32(bitval)
        cnt = _count_ge(key, cand_u ^ msb)
        prefix_u = jnp.where(cnt >= k, cand_u, prefix_u)
    return prefix_u ^ msb


def _wta_kernel(x_ref, w_ref, b_ref, out_ref, key_ref, pcnt_ref, tau_ref,
                th_ref, mb_ref):
    step = pl.program_id(0)
    t_blk = jax.lax.dot_general(
        x_ref[...], w_ref[...],
        (((1,), (1,)), ((), ())),
        preferred_element_type=jnp.float32,
    ) + b_ref[...]
    key_blk = _float_key(t_blk)
    key_ref[:, pl.ds(step * _BLOCK_N, _BLOCK_N)] = key_blk

    @pl.when(step == 0)
    def _():
        est = _kth_largest(key_blk[:, :_SUB], _SUBRANK, lo_bit=14)
        tau_ref[...] = jnp.broadcast_to(est & jnp.int32(-_SPACING),
                                        (_B, 128))
        pcnt_ref[...] = jnp.zeros((_NBOUND, _B, 128), jnp.int32)

    tau = tau_ref[:, :1]
    for j in range(_NBOUND):
        bj = tau + jnp.int32((_NBOUND // 2 - j) * _SPACING)
        p = jnp.zeros((_B, 128), jnp.int32)
        for v in range(_BLOCK_N // 128):
            p = p + (key_blk[:, v * 128:(v + 1) * 128] >= bj
                     ).astype(jnp.int32)
        pcnt_ref[j] = pcnt_ref[j] + p

    @pl.when(step == _NBLK - 1)
    def _():
        key = key_ref[...]
        tau0 = tau_ref[:, :1]
        lane = jax.lax.broadcasted_iota(jnp.int32, (_B, _NBOUND), 1)
        hist = jnp.zeros((_B, _NBOUND), jnp.int32)
        for j in range(_NBOUND):
            cj = jnp.sum(pcnt_ref[j], axis=1, keepdims=True)
            hist = jnp.where(lane == j, cj, hist)
        ok = (hist >= _K).astype(jnp.int32)
        jstar = _NBOUND - jnp.sum(ok, axis=1, keepdims=True)
        miss = jnp.any((jstar < 1) | (jstar > _NBOUND - 1))
        m = tau0 + (_NBOUND // 2 - jstar) * jnp.int32(_SPACING)

        @pl.when(jnp.logical_not(miss))
        def _refine():
            prefix = m
            for bit in range(10, -1, -1):
                cand = prefix + jnp.int32(1 << bit)
                cnt = _count_ge(key, cand)
                prefix = jnp.where(cnt >= _K, cand, prefix)
            th_ref[...] = jnp.broadcast_to(prefix, (_B, 128))

        @pl.when(miss)
        def _full():
            th_ref[...] = jnp.broadcast_to(_kth_largest(key, _K), (_B, 128))

        thresh = th_ref[:, :1]
        gt = key > thresh
        eq = key == thresh
        n_ge = jnp.sum((gt | eq).astype(jnp.int32), axis=1, keepdims=True)
        col = jax.lax.broadcasted_iota(jnp.int32, (_B, _OUT), 1)
        mb_ref[...] = jnp.full((_B, 128), _OUT, jnp.int32)

        @pl.when(jnp.any(n_ge > _K))
        def _tie():
            r = _K - (n_ge - jnp.sum(eq.astype(jnp.int32), axis=1,
                                     keepdims=True))
            mpref = jnp.zeros((_B, 1), jnp.int32)
            for bit in range(15, -1, -1):
                cand = mpref | jnp.int32(1 << bit)
                cntc = jnp.sum((eq & (col < cand)).astype(jnp.int32),
                               axis=1, keepdims=True)
                mpref = jnp.where(cntc <= r, cand, mpref)
            mb_ref[...] = jnp.broadcast_to(mpref, (_B, 128))

        accept = gt | (eq & (col < mb_ref[:, :1]))
        kmin = jnp.int32(-2147483648)
        pooled_k = jnp.max(jnp.where(accept, key, kmin), axis=0,
                           keepdims=True)
        out_ref[...] = jnp.where(pooled_k == kmin, jnp.float32(0.0),
                                 _key_float(pooled_k))


def kernel(inputs, W, b):
    out = pl.pallas_call(
        _wta_kernel,
        grid=(_NBLK,),
        in_specs=[
            pl.BlockSpec((_B, _IN), lambda i: (0, 0)),
            pl.BlockSpec((_BLOCK_N, _IN), lambda i: (i, 0)),
            pl.BlockSpec((1, _BLOCK_N), lambda i: (0, i)),
        ],
        out_specs=pl.BlockSpec((1, _OUT), lambda i: (0, 0)),
        out_shape=jax.ShapeDtypeStruct((1, _OUT), jnp.float32),
        scratch_shapes=[
            pltpu.VMEM((_B, _OUT), jnp.int32),
            pltpu.VMEM((_NBOUND, _B, 128), jnp.int32),
            pltpu.VMEM((_B, 128), jnp.int32),
            pltpu.VMEM((_B, 128), jnp.int32),
            pltpu.VMEM((_B, 128), jnp.int32),
        ],
    )(inputs, W, b.reshape(1, _OUT))
    return out.reshape(_OUT)


# speedup vs baseline: 1.5099x; 1.2185x over previous
import jax
import jax.numpy as jnp
from jax.experimental import pallas as pl
from jax.experimental.pallas import tpu as pltpu

_IN = 1024
_OUT = 32768
_K = 256
_B = 8
_BLOCK_N = 2048
_NBLK = _OUT // _BLOCK_N


def _float_key(t):
    i = jax.lax.bitcast_convert_type(t, jnp.int32)
    return jnp.where(i >= 0, i, i ^ jnp.int32(0x7FFFFFFF))


def _count_ge_tree(key, cand):
    m = (key >= cand).astype(jnp.int32)
    n = m.shape[1]
    parts = [jnp.sum(m[:, i * (n // 8):(i + 1) * (n // 8)], axis=1,
                     keepdims=True) for i in range(8)]
    return ((parts[0] + parts[1]) + (parts[2] + parts[3])) + \
           ((parts[4] + parts[5]) + (parts[6] + parts[7]))


def _wta_kernel(x_ref, w_ref, b_ref, out_ref, t_ref, mb_ref):
    step = pl.program_id(0)
    t_blk = jax.lax.dot_general(
        x_ref[...], w_ref[...],
        (((1,), (1,)), ((), ())),
        preferred_element_type=jnp.float32,
    ) + b_ref[...]
    t_ref[:, pl.ds(step * _BLOCK_N, _BLOCK_N)] = t_blk

    @pl.when(step == _NBLK - 1)
    def _():
        t = t_ref[...]
        key = _float_key(t)

        prefix_u = jnp.zeros((_B, 1), jnp.int32)
        for bit in range(31, -1, -1):
            bitval = (1 << bit) if bit < 31 else -(1 << 31)
            cand_u = prefix_u | jnp.int32(bitval)
            cand_s = cand_u ^ jnp.int32(-2147483648)
            cnt = _count_ge_tree(key, cand_s)
            prefix_u = jnp.where(cnt >= _K, cand_u, prefix_u)
        thresh = prefix_u ^ jnp.int32(-2147483648)

        gt = key > thresh
        eq = key == thresh
        n_ge = jnp.sum((gt | eq).astype(jnp.int32), axis=1, keepdims=True)

        col = jax.lax.broadcasted_iota(jnp.int32, (_B, _OUT), 1)
        mb_ref[...] = jnp.full((_B, 128), _OUT, jnp.int32)

        @pl.when(jnp.any(n_ge > _K))
        def _tie():
            r = _K - (n_ge - jnp.sum(eq.astype(jnp.int32), axis=1,
                                     keepdims=True))
            mpref = jnp.zeros((_B, 1), jnp.int32)
            for bit in range(15, -1, -1):
                cand = mpref | jnp.int32(1 << bit)
                cntc = jnp.sum((eq & (col < cand)).astype(jnp.int32),
                               axis=1, keepdims=True)
                mpref = jnp.where(cntc <= r, cand, mpref)
            mb_ref[...] = jnp.broadcast_to(mpref, (_B, 128))

        accept = gt | (eq & (col < mb_ref[:, :1]))
        neg = jnp.float32(-jnp.inf)
        pooled = jnp.max(jnp.where(accept, t, neg), axis=0, keepdims=True)
        out_ref[...] = jnp.where(pooled == neg, jnp.float32(0.0), pooled)


def kernel(inputs, W, b):
    out = pl.pallas_call(
        _wta_kernel,
        grid=(_NBLK,),
        in_specs=[
            pl.BlockSpec((_B, _IN), lambda i: (0, 0)),
            pl.BlockSpec((_BLOCK_N, _IN), lambda i: (i, 0)),
            pl.BlockSpec((1, _BLOCK_N), lambda i: (0, i)),
        ],
        out_specs=pl.BlockSpec((1, _OUT), lambda i: (0, 0)),
        out_shape=jax.ShapeDtypeStruct((1, _OUT), jnp.float32),
        scratch_shapes=[pltpu.VMEM((_B, _OUT), jnp.float32),
                        pltpu.VMEM((_B, 128), jnp.int32)],
    )(inputs, W, b.reshape(1, _OUT))
    return out.reshape(_OUT)
